# Initial kernel scaffold; baseline (speedup 1.0000x reference)
import jax
import jax.numpy as jnp
from jax.experimental import pallas as pl


def kernel(token_ids, token_table, op_table, var_table, const_table, struct_table, special_table):
    raise NotImplementedError("write your pallas kernel here")


# speedup vs baseline: 22.9475x; 22.9475x over previous
import functools

import jax
import jax.numpy as jnp
from jax import lax
from jax.experimental import pallas as pl
from jax.experimental.pallas import tpu as pltpu
from jax.experimental.pallas import tpu_sc as plsc

_NC = 2
_NS = 16
_NW = _NC * _NS
_C = 128
_NBUF = 4


@functools.partial(jax.jit, static_argnums=(2, 3))
def _gather_rows(table, idx, B, D):
  BPW = B // _NW
  NCH = BPW // _C
  G = NCH // _NBUF
  mesh = plsc.VectorSubcoreMesh(
      core_axis_name="c", subcore_axis_name="s",
      num_cores=_NC, num_subcores=_NS)

  @functools.partial(
      pl.kernel,
      out_type=jax.ShapeDtypeStruct((B, D), jnp.float32),
      mesh=mesh,
      scratch_types=[
          pltpu.VMEM((BPW,), jnp.int32),
          pltpu.VMEM((_NBUF, _C, D), jnp.float32),
          pltpu.SemaphoreType.DMA((_NBUF,)),
          pltpu.SemaphoreType.DMA((_NBUF,)),
      ],
      compiler_params=pltpu.CompilerParams(use_tc_tiling_on_sc=False),
  )
  def gather_kernel(table_hbm, idx_hbm, out_hbm, idx_v, rows_v, gsem, osem):
    wid = lax.axis_index("s") * _NC + lax.axis_index("c")
    base = wid * BPW
    pltpu.sync_copy(idx_hbm.at[pl.ds(base, BPW)], idx_v)

    def gd(j, b):
      return pltpu.make_async_copy(
          table_hbm.at[idx_v.at[pl.ds(j * _C, _C)]], rows_v.at[b], gsem.at[b])

    def od(j, b):
      return pltpu.make_async_copy(
          rows_v.at[b], out_hbm.at[pl.ds(base + j * _C, _C)], osem.at[b])

    for b in range(_NBUF):
      gd(b, b).start()
    for b in range(_NBUF):
      gd(b, b).wait()
      od(b, b).start()

    @pl.loop(1, G)
    def _(g):
      j0 = g * _NBUF
      for b in range(_NBUF):
        od(j0 - _NBUF + b, b).wait()
        gd(j0 + b, b).start()
      for b in range(_NBUF):
        gd(j0 + b, b).wait()
        od(j0 + b, b).start()

    for b in range(_NBUF):
      od((G - 1) * _NBUF + b, b).wait()

  return gather_kernel(table, idx)


def kernel(token_ids, token_table, op_table, var_table, const_table,
           struct_table, special_table):
  batch, seq = token_ids.shape
  half = token_table.shape[1]
  d = 2 * half
  cat = jnp.concatenate(
      [op_table, var_table, const_table, struct_table, special_table], axis=0)
  fused = jnp.concatenate([token_table, cat], axis=1)
  idx = token_ids.reshape(-1).astype(jnp.int32)
  out = _gather_rows(fused, idx, idx.shape[0], d)
  return out.reshape(batch, seq, d)
